# Initial kernel scaffold; baseline (speedup 1.0000x reference)
import jax
import jax.numpy as jnp
from jax.experimental import pallas as pl


def kernel(x, edge_index, edge_attr, W_e, b_e, eps, W1, b1, W2, b2, gamma, beta):
    raise NotImplementedError("write your pallas kernel here")


# speedup vs baseline: 1.6988x; 1.6988x over previous
import functools

import jax
import jax.numpy as jnp
from jax import lax
from jax.experimental import pallas as pl
from jax.experimental.pallas import tpu as pltpu
from jax.experimental.pallas import tpu_sc as plsc

N = 10000
E = 320000
D = 128
DE = 16
H = 2 * D
L = 2

NC = 2
NS = 16
NW = NC * NS
EW = E // NW
K = 80
NCHUNK = EW // K
ZROWS = 640
AGG_ROWS = NS * ZROWS


def _sc_message_aggregate(h, e, src, dst, zeros):
  mesh = plsc.VectorSubcoreMesh(core_axis_name="c", subcore_axis_name="s")

  @functools.partial(
      pl.kernel,
      mesh=mesh,
      out_type=jax.ShapeDtypeStruct((2 * N, D), jnp.float32),
      scratch_types=[
          pltpu.VMEM((K,), jnp.int32),
          pltpu.VMEM((K,), jnp.int32),
          pltpu.VMEM((K, D), jnp.float32),
          pltpu.VMEM((K, D), jnp.float32),
          pltpu.VMEM_SHARED((AGG_ROWS, D), jnp.float32),
          pltpu.SemaphoreType.DMA,
      ],
  )
  def k(h_hbm, e_hbm, src_hbm, dst_hbm, zero_hbm, out_hbm,
        src_v, dst_v, e_v, h_v, agg, sem):
    c = lax.axis_index("c")
    s = lax.axis_index("s")
    wid = c * NS + s

    pltpu.sync_copy(zero_hbm, agg.at[pl.ds(s * ZROWS, ZROWS)])
    plsc.subcore_barrier()

    base = wid * EW

    def chunk(j, carry):
      off = base + j * K
      pltpu.sync_copy(src_hbm.at[pl.ds(off, K)], src_v)
      pltpu.sync_copy(dst_hbm.at[pl.ds(off, K)], dst_v)
      pltpu.sync_copy(e_hbm.at[pl.ds(off, K)], e_v)
      pltpu.async_copy(h_hbm.at[src_v], h_v, sem).wait()

      def row(i, carry2):
        for r in range(D // 16):
          sl = pl.ds(r * 16, 16)
          h_v[i, sl] = jnp.maximum(h_v[i, sl] + e_v[i, sl], 0.0)
        return carry2

      lax.fori_loop(0, K, row, 0, unroll=2)
      pltpu.sync_copy(h_v, agg.at[dst_v], add=True)
      return carry

    lax.fori_loop(0, NCHUNK, chunk, 0)
    plsc.subcore_barrier()

    @pl.when(s < NS - 1)
    def _():
      pltpu.sync_copy(agg.at[pl.ds(s * ZROWS, ZROWS)],
                      out_hbm.at[pl.ds(c * N + s * ZROWS, ZROWS)])

    @pl.when(s == NS - 1)
    def _():
      last = N - (NS - 1) * ZROWS
      pltpu.sync_copy(agg.at[pl.ds((NS - 1) * ZROWS, last)],
                      out_hbm.at[pl.ds(c * N + (NS - 1) * ZROWS, last)])

  return k(h, e, src, dst, zeros)


_BE = 4000


def _embed_body(ea_ref, w_ref, b_ref, out_ref):
  out_ref[...] = (
      jnp.dot(ea_ref[...], w_ref[...], preferred_element_type=jnp.float32)
      + b_ref[...])


def _tc_edge_embed(edge_attr, w, b):
  return pl.pallas_call(
      _embed_body,
      grid=(E // _BE,),
      in_specs=[
          pl.BlockSpec((_BE, DE), lambda i: (i, 0)),
          pl.BlockSpec((DE, D), lambda i: (0, 0)),
          pl.BlockSpec((1, D), lambda i: (0, 0)),
      ],
      out_specs=pl.BlockSpec((_BE, D), lambda i: (i, 0)),
      out_shape=jax.ShapeDtypeStruct((E, D), jnp.float32),
  )(edge_attr, w, b.reshape(1, D))


def _mlp_bn_body(relu_out, h_ref, a0_ref, a1_ref, sc_ref, w1_ref, b1_ref,
                 w2_ref, b2_ref, g_ref, bt_ref, out_ref):
  zin = sc_ref[...] * h_ref[...] + a0_ref[...] + a1_ref[...]
  t = jnp.maximum(
      jnp.dot(zin, w1_ref[...], preferred_element_type=jnp.float32)
      + b1_ref[...], 0.0)
  z = (jnp.dot(t, w2_ref[...], preferred_element_type=jnp.float32)
       + b2_ref[...])
  mean = jnp.mean(z, axis=0, keepdims=True)
  var = jnp.mean((z - mean) ** 2, axis=0, keepdims=True)
  zn = (z - mean) * lax.rsqrt(var + 1e-5) * g_ref[...] + bt_ref[...]
  if relu_out:
    zn = jnp.maximum(zn, 0.0)
  out_ref[...] = zn


def _tc_mlp_bn(h, a0, a1, scale, w1, b1, w2, b2, gamma, beta, relu_out):
  return pl.pallas_call(
      functools.partial(_mlp_bn_body, relu_out),
      out_shape=jax.ShapeDtypeStruct((N, D), jnp.float32),
  )(h, a0, a1,
    jnp.broadcast_to(scale.reshape(1, 1), (1, D)),
    w1, b1.reshape(1, H), w2, b2.reshape(1, D),
    gamma.reshape(1, D), beta.reshape(1, D))


def kernel(x, edge_index, edge_attr, W_e, b_e, eps, W1, b1, W2, b2,
           gamma, beta):
  src = edge_index[0]
  dst = edge_index[1]
  zeros = jnp.zeros((ZROWS, D), dtype=jnp.float32)

  h = x
  for l in range(L):
    e = _tc_edge_embed(edge_attr, W_e[l], b_e[l])
    agg2 = _sc_message_aggregate(h, e, src, dst, zeros)
    h = _tc_mlp_bn(h, agg2[:N], agg2[N:], 1.0 + eps[l],
                   W1[l], b1[l], W2[l], b2[l], gamma[l], beta[l],
                   relu_out=(l < L - 1))
  return h


# speedup vs baseline: 2.7647x; 1.6274x over previous
import functools

import jax
import jax.numpy as jnp
from jax import lax
from jax.experimental import pallas as pl
from jax.experimental.pallas import tpu as pltpu
from jax.experimental.pallas import tpu_sc as plsc

N = 10000
E = 320000
D = 128
DE = 16
H = 2 * D
L = 2

NC = 2
NS = 16
NW = NC * NS
EW = E // NW
K = 40
NCHUNK = EW // K
SLOTS = 4
ZROWS = 640
AGG_ROWS = NS * ZROWS


def _sc_message_aggregate(h, e, src, dst, zeros):
  mesh = plsc.VectorSubcoreMesh(core_axis_name="c", subcore_axis_name="s")

  @functools.partial(
      pl.kernel,
      mesh=mesh,
      out_type=jax.ShapeDtypeStruct((2 * N, D), jnp.float32),
      scratch_types=[
          pltpu.VMEM((SLOTS, K), jnp.int32),
          pltpu.VMEM((SLOTS, K), jnp.int32),
          pltpu.VMEM((SLOTS, K, D), jnp.float32),
          pltpu.VMEM((SLOTS, K, D), jnp.float32),
          pltpu.VMEM_SHARED((AGG_ROWS, D), jnp.float32),
          pltpu.SemaphoreType.DMA((SLOTS,)),
          pltpu.SemaphoreType.DMA((SLOTS,)),
          pltpu.SemaphoreType.DMA((SLOTS,)),
          pltpu.SemaphoreType.DMA((SLOTS,)),
      ],
  )
  def k(h_hbm, e_hbm, src_hbm, dst_hbm, zero_hbm, out_hbm,
        src_v, dst_v, e_v, h_v, agg, sem_i, sem_e, sem_g, sem_s):
    c = lax.axis_index("c")
    s = lax.axis_index("s")
    wid = c * NS + s

    pltpu.sync_copy(zero_hbm, agg.at[pl.ds(s * ZROWS, ZROWS)])
    plsc.subcore_barrier()

    base = wid * EW

    def in_start(j, b):
      off = base + j * K
      pltpu.async_copy(src_hbm.at[pl.ds(off, K)], src_v.at[b], sem_i.at[b])
      pltpu.async_copy(dst_hbm.at[pl.ds(off, K)], dst_v.at[b], sem_i.at[b])
      pltpu.async_copy(e_hbm.at[pl.ds(off, K)], e_v.at[b], sem_e.at[b])

    def idx_wait(b):
      pltpu.make_async_copy(src_hbm.at[pl.ds(0, K)], src_v.at[b],
                            sem_i.at[b]).wait()
      pltpu.make_async_copy(dst_hbm.at[pl.ds(0, K)], dst_v.at[b],
                            sem_i.at[b]).wait()

    def e_wait(b):
      pltpu.make_async_copy(e_hbm.at[pl.ds(0, K)], e_v.at[b],
                            sem_e.at[b]).wait()

    def gather_start(b):
      pltpu.async_copy(h_hbm.at[src_v.at[b]], h_v.at[b], sem_g.at[b])

    def gather_wait(b):
      pltpu.make_async_copy(h_hbm.at[src_v.at[b]], h_v.at[b],
                            sem_g.at[b]).wait()

    def scatter_start(b):
      pltpu.async_copy(h_v.at[b], agg.at[dst_v.at[b]], sem_s.at[b], add=True)

    def scatter_wait(b):
      pltpu.make_async_copy(h_v.at[b], agg.at[dst_v.at[b]], sem_s.at[b]).wait()

    def compute(b):
      def row(i, carry2):
        for r in range(D // 16):
          sl = pl.ds(r * 16, 16)
          h_v[b, i, sl] = jnp.maximum(h_v[b, i, sl] + e_v[b, i, sl], 0.0)
        return carry2

      lax.fori_loop(0, K, row, 0, unroll=8)

    in_start(0, 0)
    in_start(1, 1)
    idx_wait(0)
    gather_start(0)

    for jj in range(SLOTS):
      b = jj % SLOTS
      if jj >= 2:
        scatter_wait((jj - 2) % SLOTS)
      in_start(jj + 2, (jj + 2) % SLOTS)
      idx_wait((jj + 1) % SLOTS)
      gather_start((jj + 1) % SLOTS)
      gather_wait(b)
      e_wait(b)
      compute(b)
      scatter_start(b)

    def group(g, carry):
      j0 = SLOTS + SLOTS * g
      for i in range(SLOTS):
        j = j0 + i
        scatter_wait((i + 2) % SLOTS)
        in_start(j + 2, (i + 2) % SLOTS)
        idx_wait((i + 1) % SLOTS)
        gather_start((i + 1) % SLOTS)
        gather_wait(i)
        e_wait(i)
        compute(i)
        scatter_start(i)
      return carry

    n_groups = (NCHUNK - SLOTS - 2) // SLOTS
    lax.fori_loop(0, n_groups, group, 0)

    b0 = (NCHUNK - 2) % SLOTS
    b1 = (NCHUNK - 1) % SLOTS
    scatter_wait((b0 + 2) % SLOTS)
    idx_wait(b1)
    gather_start(b1)
    gather_wait(b0)
    e_wait(b0)
    compute(b0)
    scatter_start(b0)
    scatter_wait((b1 + 2) % SLOTS)
    gather_wait(b1)
    e_wait(b1)
    compute(b1)
    scatter_start(b1)
    scatter_wait(b0)
    scatter_wait(b1)
    plsc.subcore_barrier()

    @pl.when(s < NS - 1)
    def _():
      pltpu.sync_copy(agg.at[pl.ds(s * ZROWS, ZROWS)],
                      out_hbm.at[pl.ds(c * N + s * ZROWS, ZROWS)])

    @pl.when(s == NS - 1)
    def _():
      last = N - (NS - 1) * ZROWS
      pltpu.sync_copy(agg.at[pl.ds((NS - 1) * ZROWS, last)],
                      out_hbm.at[pl.ds(c * N + (NS - 1) * ZROWS, last)])

  return k(h, e, src, dst, zeros)


_BE = 4000


def _embed_body(ea_ref, w_ref, b_ref, out_ref):
  out_ref[...] = (
      jnp.dot(ea_ref[...], w_ref[...], preferred_element_type=jnp.float32)
      + b_ref[...])


def _tc_edge_embed(edge_attr, w, b):
  return pl.pallas_call(
      _embed_body,
      grid=(E // _BE,),
      in_specs=[
          pl.BlockSpec((_BE, DE), lambda i: (i, 0)),
          pl.BlockSpec((DE, D), lambda i: (0, 0)),
          pl.BlockSpec((1, D), lambda i: (0, 0)),
      ],
      out_specs=pl.BlockSpec((_BE, D), lambda i: (i, 0)),
      out_shape=jax.ShapeDtypeStruct((E, D), jnp.float32),
  )(edge_attr, w, b.reshape(1, D))


def _mlp_bn_body(relu_out, h_ref, a0_ref, a1_ref, sc_ref, w1_ref, b1_ref,
                 w2_ref, b2_ref, g_ref, bt_ref, out_ref):
  zin = sc_ref[...] * h_ref[...] + a0_ref[...] + a1_ref[...]
  t = jnp.maximum(
      jnp.dot(zin, w1_ref[...], preferred_element_type=jnp.float32)
      + b1_ref[...], 0.0)
  z = (jnp.dot(t, w2_ref[...], preferred_element_type=jnp.float32)
       + b2_ref[...])
  mean = jnp.mean(z, axis=0, keepdims=True)
  var = jnp.mean((z - mean) ** 2, axis=0, keepdims=True)
  zn = (z - mean) * lax.rsqrt(var + 1e-5) * g_ref[...] + bt_ref[...]
  if relu_out:
    zn = jnp.maximum(zn, 0.0)
  out_ref[...] = zn


def _tc_mlp_bn(h, a0, a1, scale, w1, b1, w2, b2, gamma, beta, relu_out):
  return pl.pallas_call(
      functools.partial(_mlp_bn_body, relu_out),
      out_shape=jax.ShapeDtypeStruct((N, D), jnp.float32),
  )(h, a0, a1,
    jnp.broadcast_to(scale.reshape(1, 1), (1, D)),
    w1, b1.reshape(1, H), w2, b2.reshape(1, D),
    gamma.reshape(1, D), beta.reshape(1, D))


def kernel(x, edge_index, edge_attr, W_e, b_e, eps, W1, b1, W2, b2,
           gamma, beta):
  src = edge_index[0]
  dst = edge_index[1]
  zeros = jnp.zeros((ZROWS, D), dtype=jnp.float32)

  h = x
  for l in range(L):
    e = _tc_edge_embed(edge_attr, W_e[l], b_e[l])
    agg2 = _sc_message_aggregate(h, e, src, dst, zeros)
    h = _tc_mlp_bn(h, agg2[:N], agg2[N:], 1.0 + eps[l],
                   W1[l], b1[l], W2[l], b2[l], gamma[l], beta[l],
                   relu_out=(l < L - 1))
  return h
